# baseline (speedup 1.0000x reference)
import functools

import jax
import jax.numpy as jnp
from jax.experimental import pallas as pl
from jax.experimental.pallas import tpu as pltpu

N = 10000
E = 160000
D = 256
ROW_BLK = 1000


def _pool_mm_body(h_ref, w_ref, b_ref, o_ref):
    o_ref[...] = jnp.maximum(
        jnp.dot(h_ref[...], w_ref[...], preferred_element_type=jnp.float32)
        + b_ref[...], 0.0)


def _pool_mm(h, w, b):
    grid = N // ROW_BLK
    return pl.pallas_call(
        _pool_mm_body,
        grid=(grid,),
        in_specs=[
            pl.BlockSpec((ROW_BLK, D), lambda i: (i, 0)),
            pl.BlockSpec((D, D), lambda i: (0, 0)),
            pl.BlockSpec((D,), lambda i: (0,)),
        ],
        out_specs=pl.BlockSpec((ROW_BLK, D), lambda i: (i, 0)),
        out_shape=jax.ShapeDtypeStruct((N, D), jnp.float32),
    )(h, w, b)


def _out_mm_body(h_ref, hn_ref, ws_ref, wn_ref, b_ref, o_ref, *, act):
    r = (jnp.dot(h_ref[...], ws_ref[...], preferred_element_type=jnp.float32)
         + jnp.dot(hn_ref[...], wn_ref[...], preferred_element_type=jnp.float32)
         + b_ref[...])
    o_ref[...] = jnp.tanh(r) if act else r


def _out_mm(h, hn, ws, wn, b, act):
    grid = N // ROW_BLK
    return pl.pallas_call(
        functools.partial(_out_mm_body, act=act),
        grid=(grid,),
        in_specs=[
            pl.BlockSpec((ROW_BLK, D), lambda i: (i, 0)),
            pl.BlockSpec((ROW_BLK, D), lambda i: (i, 0)),
            pl.BlockSpec((D, D), lambda i: (0, 0)),
            pl.BlockSpec((D, D), lambda i: (0, 0)),
            pl.BlockSpec((D,), lambda i: (0,)),
        ],
        out_specs=pl.BlockSpec((ROW_BLK, D), lambda i: (i, 0)),
        out_shape=jax.ShapeDtypeStruct((N, D), jnp.float32),
    )(h, hn, ws, wn, b)


def _seg_max(m, src, dst):
    msgs = m[src]
    hn = jax.ops.segment_max(msgs, dst, num_segments=N)
    return jnp.where(jnp.isneginf(hn), 0.0, hn)


def kernel(x, edge_index,
           W_pool1, b_pool1, W_self1, W_neigh1, b1,
           W_pool2, b_pool2, W_self2, W_neigh2, b2,
           W_pool3, b_pool3, W_self3, W_neigh3, b3):
    src = edge_index[0]
    dst = edge_index[1]
    params = [
        (W_pool1, b_pool1, W_self1, W_neigh1, b1, True),
        (W_pool2, b_pool2, W_self2, W_neigh2, b2, True),
        (W_pool3, b_pool3, W_self3, W_neigh3, b3, False),
    ]
    h = x
    for wp, bp, ws, wn, b, act in params:
        m = _pool_mm(h, wp, bp)
        hn = _seg_max(m, src, dst)
        h = _out_mm(h, hn, ws, wn, b, act)
    return h
